# Initial kernel scaffold; baseline (speedup 1.0000x reference)
import jax
import jax.numpy as jnp
from jax.experimental import pallas as pl


def kernel(idx, emb1, emb2, W1, b1, W2, b2):
    raise NotImplementedError("write your pallas kernel here")


# speedup vs baseline: 3.4470x; 3.4470x over previous
import jax
import jax.numpy as jnp
from jax import lax
from jax.experimental import pallas as pl

NN = 10000
ND = 40
KK = 20
AL = 3.0
ROW_BLK = 80
HI_INIT = 0x3F815000


def _prep_body(e1, e2, w1, b1, w2, b2, nv1, nv2):
    dn = (((1,), (1,)), ((), ()))
    x1 = lax.dot_general(e1[...], w1[...], dn, preferred_element_type=jnp.float32)
    nv1[...] = jnp.tanh(AL * (x1 + b1[...]))
    x2 = lax.dot_general(e2[...], w2[...], dn, preferred_element_type=jnp.float32)
    nv2[...] = jnp.tanh(AL * (x2 + b2[...]))


def _main_body(nv1_t, nv2_t, nv1, nv2, noise, out):
    dn = (((1,), (1,)), ((), ()))
    a = lax.dot_general(nv1_t[...], nv2[...], dn, preferred_element_type=jnp.float32)
    b = lax.dot_general(nv2_t[...], nv1[...], dn, preferred_element_type=jnp.float32)
    adj = jnp.maximum(jnp.tanh(AL * (a - b)), 0.0)
    v = adj + noise[...]
    bits = lax.bitcast_convert_type(v, jnp.int32)
    r = bits.shape[0]
    ncol = bits.shape[1]

    lo0 = jnp.zeros((r, 1), jnp.int32)
    hi0 = jnp.full((r, 1), HI_INIT, jnp.int32)

    def vstep(_, carry):
        lo, hi = carry
        mid = lo + ((hi - lo) >> 1)
        cnt = jnp.sum((bits >= mid).astype(jnp.int32), axis=1, keepdims=True)
        ge = cnt >= KK
        return (jnp.where(ge, mid, lo), jnp.where(ge, hi, mid))

    t, _ = lax.fori_loop(0, 30, vstep, (lo0, hi0))

    c_gt = jnp.sum((bits > t).astype(jnp.int32), axis=1, keepdims=True)
    n_fill = KK - c_gt
    eq = bits == t
    col = lax.broadcasted_iota(jnp.int32, bits.shape, 1)

    jlo0 = jnp.full((r, 1), -1, jnp.int32)
    jhi0 = jnp.full((r, 1), ncol - 1, jnp.int32)

    def cstep(_, carry):
        jlo, jhi = carry
        mid = jlo + ((jhi - jlo) >> 1)
        cnt = jnp.sum((eq & (col <= mid)).astype(jnp.int32), axis=1, keepdims=True)
        ge = cnt >= n_fill
        return (jnp.where(ge, jlo, mid), jnp.where(ge, mid, jhi))

    _, jcut = lax.fori_loop(0, 14, cstep, (jlo0, jhi0))

    sel = (bits > t) | (eq & (col <= jcut))
    out[...] = jnp.where(sel, adj, 0.0)


def kernel(idx, emb1, emb2, W1, b1, W2, b2):
    e1 = jnp.take(emb1, idx, axis=0)
    e2 = jnp.take(emb2, idx, axis=0)
    nv1, nv2 = pl.pallas_call(
        _prep_body,
        out_shape=(
            jax.ShapeDtypeStruct((NN, ND), jnp.float32),
            jax.ShapeDtypeStruct((NN, ND), jnp.float32),
        ),
    )(e1, e2, W1, b1.reshape(1, ND), W2, b2.reshape(1, ND))

    noise = jax.random.uniform(jax.random.key(1234), (NN, NN), jnp.float32) * 0.01

    grid = NN // ROW_BLK
    out = pl.pallas_call(
        _main_body,
        grid=(grid,),
        in_specs=[
            pl.BlockSpec((ROW_BLK, ND), lambda i: (i, 0)),
            pl.BlockSpec((ROW_BLK, ND), lambda i: (i, 0)),
            pl.BlockSpec((NN, ND), lambda i: (0, 0)),
            pl.BlockSpec((NN, ND), lambda i: (0, 0)),
            pl.BlockSpec((ROW_BLK, NN), lambda i: (i, 0)),
        ],
        out_specs=pl.BlockSpec((ROW_BLK, NN), lambda i: (i, 0)),
        out_shape=jax.ShapeDtypeStruct((NN, NN), jnp.float32),
    )(nv1, nv2, nv1, nv2, noise)
    return out


# speedup vs baseline: 4.1358x; 1.1998x over previous
import jax
import jax.numpy as jnp
from jax import lax
from jax.experimental import pallas as pl

NN = 10000
ND = 40
KK = 20
AL = 3.0
ROW_BLK = 80
ONE_BITS = 0x3F800000


def _prep_body(e1, e2, w1, b1, w2, b2, nv1, nv2):
    dn = (((1,), (1,)), ((), ()))
    x1 = lax.dot_general(e1[...], w1[...], dn, preferred_element_type=jnp.float32)
    nv1[...] = jnp.tanh(AL * (x1 + b1[...]))
    x2 = lax.dot_general(e2[...], w2[...], dn, preferred_element_type=jnp.float32)
    nv2[...] = jnp.tanh(AL * (x2 + b2[...]))


def _main_body(nv1_t, nv2_t, nv1, nv2, noise, out):
    dn = (((1,), (1,)), ((), ()))
    a = lax.dot_general(nv1_t[...], nv2[...], dn, preferred_element_type=jnp.float32)
    b = lax.dot_general(nv2_t[...], nv1[...], dn, preferred_element_type=jnp.float32)
    adj = jnp.maximum(jnp.tanh(AL * (a - b)), 0.0)
    v = adj + noise[...]
    bits = lax.bitcast_convert_type(v, jnp.int32)
    r = bits.shape[0]
    ncol = bits.shape[1]

    def cnt_ge(th):
        return jnp.sum((bits >= th).astype(jnp.int32), axis=1, keepdims=True)

    one = jnp.int32(ONE_BITS)
    hi0 = jnp.max(bits, axis=1, keepdims=True) + 1
    lo0 = jnp.where(cnt_ge(one) >= KK, one, 0)
    cgt0 = jnp.zeros((r, 1), jnp.int32)

    def vcond(carry):
        lo, hi, _ = carry
        return jnp.max(hi - lo) > 1

    def vstep(carry):
        lo, hi, cgt = carry
        mid = lo + ((hi - lo) >> 1)
        cnt = cnt_ge(mid)
        ge = cnt >= KK
        return (jnp.where(ge, mid, lo), jnp.where(ge, hi, mid),
                jnp.where(ge, cgt, cnt))

    t, _, c_gt = lax.while_loop(vcond, vstep, (lo0, hi0, cgt0))
    n_fill = KK - c_gt
    eq = bits == t
    col = lax.broadcasted_iota(jnp.int32, bits.shape, 1)
    ec = jnp.where(eq, col, jnp.int32(0x7FFFFFFF))

    jlo0 = jnp.full((r, 1), -1, jnp.int32)
    jhi0 = jnp.full((r, 1), ncol - 1, jnp.int32)

    def cstep(_, carry):
        jlo, jhi = carry
        mid = jlo + ((jhi - jlo) >> 1)
        cnt = jnp.sum((ec <= mid).astype(jnp.int32), axis=1, keepdims=True)
        ge = cnt >= n_fill
        return (jnp.where(ge, jlo, mid), jnp.where(ge, mid, jhi))

    _, jcut = lax.fori_loop(0, 14, cstep, (jlo0, jhi0))

    sel = (bits > t) | (ec <= jcut)
    out[...] = jnp.where(sel, adj, 0.0)


def kernel(idx, emb1, emb2, W1, b1, W2, b2):
    e1 = jnp.take(emb1, idx, axis=0)
    e2 = jnp.take(emb2, idx, axis=0)
    nv1, nv2 = pl.pallas_call(
        _prep_body,
        out_shape=(
            jax.ShapeDtypeStruct((NN, ND), jnp.float32),
            jax.ShapeDtypeStruct((NN, ND), jnp.float32),
        ),
    )(e1, e2, W1, b1.reshape(1, ND), W2, b2.reshape(1, ND))

    noise = jax.random.uniform(jax.random.key(1234), (NN, NN), jnp.float32) * 0.01

    grid = NN // ROW_BLK
    out = pl.pallas_call(
        _main_body,
        grid=(grid,),
        in_specs=[
            pl.BlockSpec((ROW_BLK, ND), lambda i: (i, 0)),
            pl.BlockSpec((ROW_BLK, ND), lambda i: (i, 0)),
            pl.BlockSpec((NN, ND), lambda i: (0, 0)),
            pl.BlockSpec((NN, ND), lambda i: (0, 0)),
            pl.BlockSpec((ROW_BLK, NN), lambda i: (i, 0)),
        ],
        out_specs=pl.BlockSpec((ROW_BLK, NN), lambda i: (i, 0)),
        out_shape=jax.ShapeDtypeStruct((NN, NN), jnp.float32),
    )(nv1, nv2, nv1, nv2, noise)
    return out


# speedup vs baseline: 4.1897x; 1.0130x over previous
import jax
import jax.numpy as jnp
from jax import lax
from jax.experimental import pallas as pl

NN = 10000
ND = 40
KK = 20
AL = 3.0
ROW_BLK = 80
ONE_BITS = 0x3F800000


def _prep_body(e1, e2, w1, b1, w2, b2, nv1, nv2):
    dn = (((1,), (1,)), ((), ()))
    x1 = lax.dot_general(e1[...], w1[...], dn, preferred_element_type=jnp.float32)
    nv1[...] = jnp.tanh(AL * (x1 + b1[...]))
    x2 = lax.dot_general(e2[...], w2[...], dn, preferred_element_type=jnp.float32)
    nv2[...] = jnp.tanh(AL * (x2 + b2[...]))


def _rotl(x, d):
    return lax.shift_left(x, jnp.int32(d)) | lax.shift_right_logical(
        x, jnp.int32(32 - d))


def _noise_bits(flat):
    ks0 = jnp.int32(0)
    ks1 = jnp.int32(1234)
    ks2 = ks0 ^ ks1 ^ jnp.int32(0x1BD11BDA)
    rot_a = (13, 15, 26, 6)
    rot_b = (17, 29, 16, 24)

    def rounds(x0, x1, rots):
        for rr in rots:
            x0 = x0 + x1
            x1 = x0 ^ _rotl(x1, rr)
        return x0, x1

    x0 = jnp.zeros_like(flat) + ks0
    x1 = flat + ks1
    x0, x1 = rounds(x0, x1, rot_a)
    x0 = x0 + ks1
    x1 = x1 + (ks2 + jnp.int32(1))
    x0, x1 = rounds(x0, x1, rot_b)
    x0 = x0 + ks2
    x1 = x1 + (ks0 + jnp.int32(2))
    x0, x1 = rounds(x0, x1, rot_a)
    x0 = x0 + ks0
    x1 = x1 + (ks1 + jnp.int32(3))
    x0, x1 = rounds(x0, x1, rot_b)
    x0 = x0 + ks1
    x1 = x1 + (ks2 + jnp.int32(4))
    x0, x1 = rounds(x0, x1, rot_a)
    x0 = x0 + ks2
    x1 = x1 + (ks0 + jnp.int32(5))
    return x0 ^ x1


def _main_body(nv1_t, nv2_t, nv1, nv2, out):
    dn = (((1,), (1,)), ((), ()))
    a = lax.dot_general(nv1_t[...], nv2[...], dn, preferred_element_type=jnp.float32)
    b = lax.dot_general(nv2_t[...], nv1[...], dn, preferred_element_type=jnp.float32)
    adj = jnp.maximum(jnp.tanh(AL * (a - b)), 0.0)
    row0 = pl.program_id(0) * a.shape[0]
    flat = ((row0 + lax.broadcasted_iota(jnp.int32, a.shape, 0)) * NN
            + lax.broadcasted_iota(jnp.int32, a.shape, 1))
    rb = _noise_bits(flat)
    fb = lax.shift_right_logical(rb, jnp.int32(9)) | jnp.int32(0x3F800000)
    u = lax.bitcast_convert_type(fb, jnp.float32) - jnp.float32(1.0)
    v = adj + u * jnp.float32(0.01)
    bits = lax.bitcast_convert_type(v, jnp.int32)
    r = bits.shape[0]
    ncol = bits.shape[1]

    def cnt_ge(th):
        return jnp.sum((bits >= th).astype(jnp.int32), axis=1, keepdims=True)

    one = jnp.int32(ONE_BITS)
    hi0 = jnp.max(bits, axis=1, keepdims=True) + 1
    lo0 = jnp.where(cnt_ge(one) >= KK, one, 0)
    cgt0 = jnp.zeros((r, 1), jnp.int32)

    def vcond(carry):
        lo, hi, _ = carry
        return jnp.max(hi - lo) > 1

    def vstep(carry):
        lo, hi, cgt = carry
        mid = lo + ((hi - lo) >> 1)
        cnt = cnt_ge(mid)
        ge = cnt >= KK
        return (jnp.where(ge, mid, lo), jnp.where(ge, hi, mid),
                jnp.where(ge, cgt, cnt))

    t, _, c_gt = lax.while_loop(vcond, vstep, (lo0, hi0, cgt0))
    n_fill = KK - c_gt
    eq = bits == t
    col = lax.broadcasted_iota(jnp.int32, bits.shape, 1)
    ec = jnp.where(eq, col, jnp.int32(0x7FFFFFFF))

    jlo0 = jnp.full((r, 1), -1, jnp.int32)
    jhi0 = jnp.full((r, 1), ncol - 1, jnp.int32)

    def cstep(_, carry):
        jlo, jhi = carry
        mid = jlo + ((jhi - jlo) >> 1)
        cnt = jnp.sum((ec <= mid).astype(jnp.int32), axis=1, keepdims=True)
        ge = cnt >= n_fill
        return (jnp.where(ge, jlo, mid), jnp.where(ge, mid, jhi))

    _, jcut = lax.fori_loop(0, 14, cstep, (jlo0, jhi0))

    sel = (bits > t) | (ec <= jcut)
    out[...] = jnp.where(sel, adj, 0.0)


def kernel(idx, emb1, emb2, W1, b1, W2, b2):
    e1 = jnp.take(emb1, idx, axis=0)
    e2 = jnp.take(emb2, idx, axis=0)
    nv1, nv2 = pl.pallas_call(
        _prep_body,
        out_shape=(
            jax.ShapeDtypeStruct((NN, ND), jnp.float32),
            jax.ShapeDtypeStruct((NN, ND), jnp.float32),
        ),
    )(e1, e2, W1, b1.reshape(1, ND), W2, b2.reshape(1, ND))

    grid = NN // ROW_BLK
    out = pl.pallas_call(
        _main_body,
        grid=(grid,),
        in_specs=[
            pl.BlockSpec((ROW_BLK, ND), lambda i: (i, 0)),
            pl.BlockSpec((ROW_BLK, ND), lambda i: (i, 0)),
            pl.BlockSpec((NN, ND), lambda i: (0, 0)),
            pl.BlockSpec((NN, ND), lambda i: (0, 0)),
        ],
        out_specs=pl.BlockSpec((ROW_BLK, NN), lambda i: (i, 0)),
        out_shape=jax.ShapeDtypeStruct((NN, NN), jnp.float32),
    )(nv1, nv2, nv1, nv2)
    return out


# speedup vs baseline: 4.2080x; 1.0044x over previous
import functools

import jax
import jax.numpy as jnp
from jax import lax
from jax.experimental import pallas as pl
from jax.experimental.pallas import tpu as pltpu
from jax.experimental.pallas import tpu_sc as plsc

NN = 10000
ND = 40
KK = 20
AL = 3.0
ROW_BLK = 80
ONE_BITS = 0x3F800000


def _prep_body(e1, e2, w1, b1, w2, b2, nv1, nv2):
    dn = (((1,), (1,)), ((), ()))
    x1 = lax.dot_general(e1[...], w1[...], dn, preferred_element_type=jnp.float32)
    nv1[...] = jnp.tanh(AL * (x1 + b1[...]))
    x2 = lax.dot_general(e2[...], w2[...], dn, preferred_element_type=jnp.float32)
    nv2[...] = jnp.tanh(AL * (x2 + b2[...]))


def _rotl(x, d):
    return lax.shift_left(x, jnp.int32(d)) | lax.shift_right_logical(
        x, jnp.int32(32 - d))


def _noise_bits(flat):
    ks0 = jnp.int32(0)
    ks1 = jnp.int32(1234)
    ks2 = ks0 ^ ks1 ^ jnp.int32(0x1BD11BDA)
    rot_a = (13, 15, 26, 6)
    rot_b = (17, 29, 16, 24)

    def rounds(x0, x1, rots):
        for rr in rots:
            x0 = x0 + x1
            x1 = x0 ^ _rotl(x1, rr)
        return x0, x1

    x0 = jnp.zeros_like(flat) + ks0
    x1 = flat + ks1
    x0, x1 = rounds(x0, x1, rot_a)
    x0 = x0 + ks1
    x1 = x1 + (ks2 + jnp.int32(1))
    x0, x1 = rounds(x0, x1, rot_b)
    x0 = x0 + ks2
    x1 = x1 + (ks0 + jnp.int32(2))
    x0, x1 = rounds(x0, x1, rot_a)
    x0 = x0 + ks0
    x1 = x1 + (ks1 + jnp.int32(3))
    x0, x1 = rounds(x0, x1, rot_b)
    x0 = x0 + ks1
    x1 = x1 + (ks2 + jnp.int32(4))
    x0, x1 = rounds(x0, x1, rot_a)
    x0 = x0 + ks2
    x1 = x1 + (ks0 + jnp.int32(5))
    return x0 ^ x1


def _main_body(nv1_t, nv2_t, nv1, nv2, out):
    dn = (((1,), (1,)), ((), ()))
    a = lax.dot_general(nv1_t[...], nv2[...], dn, preferred_element_type=jnp.float32)
    b = lax.dot_general(nv2_t[...], nv1[...], dn, preferred_element_type=jnp.float32)
    adj = jnp.maximum(jnp.tanh(AL * (a - b)), 0.0)
    row0 = pl.program_id(0) * a.shape[0]
    flat = ((row0 + lax.broadcasted_iota(jnp.int32, a.shape, 0)) * NN
            + lax.broadcasted_iota(jnp.int32, a.shape, 1))
    rb = _noise_bits(flat)
    fb = lax.shift_right_logical(rb, jnp.int32(9)) | jnp.int32(0x3F800000)
    u = lax.bitcast_convert_type(fb, jnp.float32) - jnp.float32(1.0)
    v = adj + u * jnp.float32(0.01)
    bits = lax.bitcast_convert_type(v, jnp.int32)
    r = bits.shape[0]
    ncol = bits.shape[1]

    def cnt_ge(th):
        return jnp.sum((bits >= th).astype(jnp.int32), axis=1, keepdims=True)

    one = jnp.int32(ONE_BITS)
    hi0 = jnp.max(bits, axis=1, keepdims=True) + 1
    lo0 = jnp.where(cnt_ge(one) >= KK, one, 0)
    cgt0 = jnp.zeros((r, 1), jnp.int32)

    def vcond(carry):
        lo, hi, _ = carry
        return jnp.max(hi - lo) > 1

    def vstep(carry):
        lo, hi, cgt = carry
        mid = lo + ((hi - lo) >> 1)
        cnt = cnt_ge(mid)
        ge = cnt >= KK
        return (jnp.where(ge, mid, lo), jnp.where(ge, hi, mid),
                jnp.where(ge, cgt, cnt))

    t, _, c_gt = lax.while_loop(vcond, vstep, (lo0, hi0, cgt0))
    n_fill = KK - c_gt
    eq = bits == t
    col = lax.broadcasted_iota(jnp.int32, bits.shape, 1)
    ec = jnp.where(eq, col, jnp.int32(0x7FFFFFFF))

    jlo0 = jnp.full((r, 1), -1, jnp.int32)
    jhi0 = jnp.full((r, 1), ncol - 1, jnp.int32)

    def cstep(_, carry):
        jlo, jhi = carry
        mid = jlo + ((jhi - jlo) >> 1)
        cnt = jnp.sum((ec <= mid).astype(jnp.int32), axis=1, keepdims=True)
        ge = cnt >= n_fill
        return (jnp.where(ge, jlo, mid), jnp.where(ge, mid, jhi))

    _, jcut = lax.fori_loop(0, 14, cstep, (jlo0, jhi0))

    sel = (bits > t) | (ec <= jcut)
    out[...] = jnp.where(sel, adj, 0.0)


_SC_BPW = 400
_SC_NW = NN // _SC_BPW
_SC_D = 128


def _sc_gather_two(emb1, emb2, idx):
    mesh = plsc.VectorSubcoreMesh(core_axis_name="c", subcore_axis_name="s")

    @functools.partial(
        pl.kernel,
        mesh=mesh,
        out_type=(
            jax.ShapeDtypeStruct((NN, _SC_D), jnp.float32),
            jax.ShapeDtypeStruct((NN, _SC_D), jnp.float32),
        ),
        scratch_types=[
            pltpu.VMEM((_SC_BPW,), jnp.int32),
            pltpu.VMEM((_SC_BPW, _SC_D), jnp.float32),
            pltpu.VMEM((_SC_BPW, _SC_D), jnp.float32),
            pltpu.SemaphoreType.DMA,
            pltpu.SemaphoreType.DMA,
        ],
    )
    def k(e1_hbm, e2_hbm, idx_hbm, o1_hbm, o2_hbm, idx_v, r1_v, r2_v, s1, s2):
        wid = lax.axis_index("s") * 2 + lax.axis_index("c")

        @pl.when(wid < _SC_NW)
        def _():
            base = wid * _SC_BPW
            pltpu.sync_copy(idx_hbm.at[pl.ds(base, _SC_BPW)], idx_v)
            c1 = pltpu.async_copy(e1_hbm.at[idx_v], r1_v, s1)
            c2 = pltpu.async_copy(e2_hbm.at[idx_v], r2_v, s2)
            c1.wait()
            c2.wait()
            pltpu.sync_copy(r1_v, o1_hbm.at[pl.ds(base, _SC_BPW)])
            pltpu.sync_copy(r2_v, o2_hbm.at[pl.ds(base, _SC_BPW)])

    pad = ((0, 0), (0, _SC_D - ND))
    g1, g2 = k(jnp.pad(emb1, pad), jnp.pad(emb2, pad), idx)
    return g1[:, :ND], g2[:, :ND]


def kernel(idx, emb1, emb2, W1, b1, W2, b2):
    e1, e2 = _sc_gather_two(emb1, emb2, idx)
    nv1, nv2 = pl.pallas_call(
        _prep_body,
        out_shape=(
            jax.ShapeDtypeStruct((NN, ND), jnp.float32),
            jax.ShapeDtypeStruct((NN, ND), jnp.float32),
        ),
    )(e1, e2, W1, b1.reshape(1, ND), W2, b2.reshape(1, ND))

    grid = NN // ROW_BLK
    out = pl.pallas_call(
        _main_body,
        grid=(grid,),
        in_specs=[
            pl.BlockSpec((ROW_BLK, ND), lambda i: (i, 0)),
            pl.BlockSpec((ROW_BLK, ND), lambda i: (i, 0)),
            pl.BlockSpec((NN, ND), lambda i: (0, 0)),
            pl.BlockSpec((NN, ND), lambda i: (0, 0)),
        ],
        out_specs=pl.BlockSpec((ROW_BLK, NN), lambda i: (i, 0)),
        out_shape=jax.ShapeDtypeStruct((NN, NN), jnp.float32),
    )(nv1, nv2, nv1, nv2)
    return out
